# baseline (speedup 1.0000x reference)
import jax
import jax.numpy as jnp
from jax import lax
from jax.experimental import pallas as pl
from jax.experimental.pallas import tpu as pltpu
from jax.experimental.pallas import tpu_sc as plsc
import functools

NUM_EMB = 1_000_000
DIM = 64
PDIM = 128
BATCH = 16384
FIELDS = 26
FPAD = 32
TOT = BATCH * FIELDS

NC = 2
NS = 16
NW = NC * NS
L = 16

GCOLS = 256
NGRP = NUM_EMB // GCOLS
TAIL = NUM_EMB - NGRP * GCOLS
assert TAIL == 64

BPW = BATCH // NW
LPW = BPW * FIELDS
SUB = 16
CHUNK = SUB * FIELDS
NCHUNK = LPW // CHUNK
assert NCHUNK % 2 == 0

_mesh = plsc.VectorSubcoreMesh(
    core_axis_name="c", subcore_axis_name="s", num_cores=NC, num_subcores=NS
)


@functools.partial(
    pl.kernel,
    out_type=jax.ShapeDtypeStruct((NUM_EMB, PDIM), jnp.float32),
    mesh=_mesh,
    scratch_types=[
        pltpu.VMEM((DIM, GCOLS), jnp.float32),
        pltpu.VMEM((DIM, GCOLS), jnp.float32),
        pltpu.VMEM((GCOLS, PDIM), jnp.float32),
        pltpu.VMEM((GCOLS, PDIM), jnp.float32),
        pltpu.SemaphoreType.DMA,
        pltpu.SemaphoreType.DMA,
        pltpu.SemaphoreType.DMA,
        pltpu.SemaphoreType.DMA,
    ],
    compiler_params=pltpu.CompilerParams(needs_layout_passes=False),
)
def _sc_repack(wt, tblp, in0, in1, ot0, ot1, si0, si1, so0, so1):
    wid = lax.axis_index("s") * NC + lax.axis_index("c")
    ins = (in0, in1)
    ots = (ot0, ot1)
    sis = (si0, si1)
    sos = (so0, so1)
    lanes = lax.iota(jnp.int32, L)

    g_lo = wid * NGRP // NW
    g_hi = (wid + 1) * NGRP // NW
    cnt = g_hi - g_lo

    def load(g, p, w):
        @pl.loop(0, DIM // 8)
        def _(t):
            pltpu.make_async_copy(
                wt.at[pl.ds(t * 8, 8), pl.ds(g * GCOLS, w)],
                ins[p].at[pl.ds(t * 8, 8), pl.ds(0, w)],
                sis[p],
            ).start()

    def loadwait(p, w):
        pltpu.make_async_copy(
            wt.at[pl.ds(0, DIM), pl.ds(0, w)], ins[p].at[:, pl.ds(0, w)], sis[p]
        ).wait()

    def transpose(p, n):
        @pl.loop(0, n // L)
        def _(rq):
            for d in range(DIM):
                vec = ins[p][d, pl.ds(rq * L, L)]
                plsc.store_scatter(
                    ots[p], [rq * L + lanes, jnp.full((L,), d, jnp.int32)], vec
                )

    def store(g, p, n):
        pltpu.make_async_copy(
            ots[p].at[pl.ds(0, n), :], tblp.at[pl.ds(g * GCOLS, n), :], sos[p]
        ).start()

    def storewait(p, n):
        pltpu.make_async_copy(
            ots[p].at[pl.ds(0, n), :], tblp.at[pl.ds(0, n), :], sos[p]
        ).wait()

    def body(g, p):
        @pl.when(g + 1 < g_hi)
        def _():
            load(g + 1, 1 - p, GCOLS)

        loadwait(p, GCOLS)

        @pl.when(g - g_lo >= 2)
        def _():
            storewait(p, GCOLS)

        transpose(p, GCOLS)
        store(g, p, GCOLS)

    load(g_lo, 0, GCOLS)

    @pl.loop(0, cnt // 2)
    def _(jp):
        g0 = g_lo + 2 * jp
        body(g0, 0)
        body(g0 + 1, 1)

    @pl.when(cnt % 2 == 1)
    def _():
        body(g_hi - 1, 0)

    storewait(0, GCOLS)
    storewait(1, GCOLS)

    @pl.when(wid == NW - 1)
    def _():
        gt = jnp.int32(NGRP)
        load(gt, 0, PDIM)
        loadwait(0, PDIM)
        transpose(0, TAIL)
        store(gt, 0, TAIL)
        storewait(0, TAIL)


@functools.partial(
    pl.kernel,
    out_type=jax.ShapeDtypeStruct((BATCH * FPAD, PDIM), jnp.float32),
    mesh=_mesh,
    scratch_types=[
        pltpu.VMEM((LPW,), jnp.int32),
        pltpu.VMEM((CHUNK, PDIM), jnp.float32),
        pltpu.VMEM((CHUNK, PDIM), jnp.float32),
        pltpu.SemaphoreType.DMA,
        pltpu.SemaphoreType.DMA,
    ],
    compiler_params=pltpu.CompilerParams(use_tc_tiling_on_sc=False),
)
def _sc_gather(tbl, idx, out, idx_v, rows0, rows1, sem0, sem1):
    wid = lax.axis_index("s") * NC + lax.axis_index("c")
    base = wid * LPW
    pltpu.sync_copy(idx.at[pl.ds(base, LPW)], idx_v)

    bufs = (rows0, rows1)
    sems = (sem0, sem1)

    def start(c, b):
        pltpu.make_async_copy(
            tbl.at[idx_v.at[pl.ds(c * CHUNK, CHUNK)]], bufs[b], sems[b]
        ).start()

    def finish(c, b):
        pltpu.make_async_copy(
            tbl.at[idx_v.at[pl.ds(c * CHUNK, CHUNK)]], bufs[b], sems[b]
        ).wait()
        b0 = wid * BPW + c * SUB
        for j in range(SUB):
            pltpu.sync_copy(
                bufs[b].at[pl.ds(j * FIELDS, FIELDS), :],
                out.at[pl.ds((b0 + j) * FPAD, FIELDS), :],
            )

    start(0, 0)
    start(1, 1)

    @pl.loop(0, NCHUNK // 2 - 1)
    def _(g):
        for b in range(2):
            c = 2 * g + b
            finish(c, b)
            start(c + 2, b)

    for b in range(2):
        finish(NCHUNK - 2 + b, b)


def kernel(input, weight):
    idx = input.reshape(-1).astype(jnp.int32)
    tblp = _sc_repack(weight.T)
    out = _sc_gather(tblp, idx)
    return out.reshape(BATCH, FPAD, PDIM)[:, :FIELDS, :DIM]


# speedup vs baseline: 1.2472x; 1.2472x over previous
import jax
import jax.numpy as jnp
from jax import lax
from jax.experimental import pallas as pl
from jax.experimental.pallas import tpu as pltpu
from jax.experimental.pallas import tpu_sc as plsc
import functools

NUM_EMB = 1_000_000
DIM = 64
PDIM = 128
BATCH = 16384
FIELDS = 26
FPAD = 32
TOT = BATCH * FIELDS

NC = 2
NS = 16
NW = NC * NS
L = 16

GCOLS = 256
NGRP = NUM_EMB // GCOLS
TAIL = NUM_EMB - NGRP * GCOLS
assert TAIL == 64

BPW = BATCH // NW
LPW = BPW * FIELDS
SUB = 16
CHUNK = SUB * FIELDS
NCHUNK = LPW // CHUNK
assert NCHUNK % 2 == 0

_mesh = plsc.VectorSubcoreMesh(
    core_axis_name="c", subcore_axis_name="s", num_cores=NC, num_subcores=NS
)


@functools.partial(
    pl.kernel,
    out_type=jax.ShapeDtypeStruct((NUM_EMB, PDIM), jnp.float32),
    mesh=_mesh,
    scratch_types=[
        pltpu.VMEM((DIM, GCOLS), jnp.float32),
        pltpu.VMEM((DIM, GCOLS), jnp.float32),
        pltpu.VMEM((GCOLS, PDIM), jnp.float32),
        pltpu.VMEM((GCOLS, PDIM), jnp.float32),
        pltpu.SemaphoreType.DMA,
        pltpu.SemaphoreType.DMA,
        pltpu.SemaphoreType.DMA,
        pltpu.SemaphoreType.DMA,
    ],
    compiler_params=pltpu.CompilerParams(needs_layout_passes=False),
)
def _sc_repack(wt, tblp, in0, in1, ot0, ot1, si0, si1, so0, so1):
    wid = lax.axis_index("s") * NC + lax.axis_index("c")
    ins = (in0, in1)
    ots = (ot0, ot1)
    sis = (si0, si1)
    sos = (so0, so1)
    lanes = lax.iota(jnp.int32, L)

    g_lo = wid * NGRP // NW
    g_hi = (wid + 1) * NGRP // NW
    cnt = g_hi - g_lo

    def load(g, p, w):
        @pl.loop(0, DIM // 8)
        def _(t):
            pltpu.make_async_copy(
                wt.at[pl.ds(t * 8, 8), pl.ds(g * GCOLS, w)],
                ins[p].at[pl.ds(t * 8, 8), pl.ds(0, w)],
                sis[p],
            ).start()

    def loadwait(p, w):
        pltpu.make_async_copy(
            wt.at[pl.ds(0, DIM), pl.ds(0, w)], ins[p].at[:, pl.ds(0, w)], sis[p]
        ).wait()

    def transpose(p, n):
        @pl.loop(0, n // L)
        def _(rq):
            rows = rq * L + lanes
            for d0 in range(0, DIM, 8):
                vecs = [ins[p][d0 + i, pl.ds(rq * L, L)] for i in range(8)]
                for i in range(8):
                    plsc.store_scatter(
                        ots[p], [rows, jnp.full((L,), d0 + i, jnp.int32)], vecs[i]
                    )

    def store(g, p, n):
        pltpu.make_async_copy(
            ots[p].at[pl.ds(0, n), :], tblp.at[pl.ds(g * GCOLS, n), :], sos[p]
        ).start()

    def storewait(p, n):
        pltpu.make_async_copy(
            ots[p].at[pl.ds(0, n), :], tblp.at[pl.ds(0, n), :], sos[p]
        ).wait()

    def body(g, p):
        @pl.when(g + 1 < g_hi)
        def _():
            load(g + 1, 1 - p, GCOLS)

        loadwait(p, GCOLS)

        @pl.when(g - g_lo >= 2)
        def _():
            storewait(p, GCOLS)

        transpose(p, GCOLS)
        store(g, p, GCOLS)

    load(g_lo, 0, GCOLS)

    @pl.loop(0, cnt // 2)
    def _(jp):
        g0 = g_lo + 2 * jp
        body(g0, 0)
        body(g0 + 1, 1)

    @pl.when(cnt % 2 == 1)
    def _():
        body(g_hi - 1, 0)

    storewait(0, GCOLS)
    storewait(1, GCOLS)

    @pl.when(wid == NW - 1)
    def _():
        gt = jnp.int32(NGRP)
        load(gt, 0, PDIM)
        loadwait(0, PDIM)
        transpose(0, TAIL)
        store(gt, 0, TAIL)
        storewait(0, TAIL)


@functools.partial(
    pl.kernel,
    out_type=jax.ShapeDtypeStruct((BATCH * FPAD, PDIM), jnp.float32),
    mesh=_mesh,
    scratch_types=[
        pltpu.VMEM((LPW,), jnp.int32),
        pltpu.VMEM((CHUNK, PDIM), jnp.float32),
        pltpu.VMEM((CHUNK, PDIM), jnp.float32),
        pltpu.SemaphoreType.DMA,
        pltpu.SemaphoreType.DMA,
    ],
    compiler_params=pltpu.CompilerParams(use_tc_tiling_on_sc=False),
)
def _sc_gather(tbl, idx, out, idx_v, rows0, rows1, sem0, sem1):
    wid = lax.axis_index("s") * NC + lax.axis_index("c")
    base = wid * LPW
    pltpu.sync_copy(idx.at[pl.ds(base, LPW)], idx_v)

    bufs = (rows0, rows1)
    sems = (sem0, sem1)

    def start(c, b):
        pltpu.make_async_copy(
            tbl.at[idx_v.at[pl.ds(c * CHUNK, CHUNK)]], bufs[b], sems[b]
        ).start()

    def finish(c, b):
        pltpu.make_async_copy(
            tbl.at[idx_v.at[pl.ds(c * CHUNK, CHUNK)]], bufs[b], sems[b]
        ).wait()
        b0 = wid * BPW + c * SUB
        for j in range(SUB):
            pltpu.sync_copy(
                bufs[b].at[pl.ds(j * FIELDS, FIELDS), :],
                out.at[pl.ds((b0 + j) * FPAD, FIELDS), :],
            )

    start(0, 0)
    start(1, 1)

    @pl.loop(0, NCHUNK // 2 - 1)
    def _(g):
        for b in range(2):
            c = 2 * g + b
            finish(c, b)
            start(c + 2, b)

    for b in range(2):
        finish(NCHUNK - 2 + b, b)


def kernel(input, weight):
    idx = input.reshape(-1).astype(jnp.int32)
    tblp = _sc_repack(weight.T)
    out = _sc_gather(tblp, idx)
    return out.reshape(BATCH, FPAD, PDIM)[:, :FIELDS, :DIM]


# speedup vs baseline: 1.9998x; 1.6035x over previous
import jax
import jax.numpy as jnp
from jax import lax
from jax.experimental import pallas as pl
from jax.experimental.pallas import tpu as pltpu
from jax.experimental.pallas import tpu_sc as plsc
import functools

NUM_EMB = 1_000_000
DIM = 64
PDIM = 128
BATCH = 16384
FIELDS = 26
FPAD = 32
TOT = BATCH * FIELDS

NC = 2
NS = 16
NW = NC * NS
L = 16

GCOLS = 256
NGRP = NUM_EMB // GCOLS
TAIL = NUM_EMB - NGRP * GCOLS
assert TAIL == 64

BPW = BATCH // NW
LPW = BPW * FIELDS
SUB = 16
CHUNK = SUB * FIELDS
NCHUNK = LPW // CHUNK
assert NCHUNK % 2 == 0

_mesh = plsc.VectorSubcoreMesh(
    core_axis_name="c", subcore_axis_name="s", num_cores=NC, num_subcores=NS
)


@functools.partial(
    pl.kernel,
    out_type=jax.ShapeDtypeStruct((NUM_EMB, PDIM), jnp.float32),
    mesh=_mesh,
    scratch_types=[
        pltpu.VMEM((DIM, GCOLS), jnp.float32),
        pltpu.VMEM((DIM, GCOLS), jnp.float32),
        pltpu.VMEM((GCOLS, PDIM), jnp.float32),
        pltpu.VMEM((GCOLS, PDIM), jnp.float32),
        pltpu.VMEM((DIM * L,), jnp.int32),
        pltpu.SemaphoreType.DMA,
        pltpu.SemaphoreType.DMA,
        pltpu.SemaphoreType.DMA,
        pltpu.SemaphoreType.DMA,
    ],
    compiler_params=pltpu.CompilerParams(needs_layout_passes=False),
)
def _sc_repack(wt, tblp, in0, in1, ot0, ot1, dtab, si0, si1, so0, so1):
    wid = lax.axis_index("s") * NC + lax.axis_index("c")
    ins = (in0, in1)
    ots = (ot0, ot1)
    sis = (si0, si1)
    sos = (so0, so1)
    lanes = lax.iota(jnp.int32, L)

    g_lo = wid * NGRP // NW
    g_hi = (wid + 1) * NGRP // NW
    cnt = g_hi - g_lo

    def load(g, p, w):
        @pl.loop(0, DIM // 8)
        def _(t):
            pltpu.make_async_copy(
                wt.at[pl.ds(t * 8, 8), pl.ds(g * GCOLS, w)],
                ins[p].at[pl.ds(t * 8, 8), pl.ds(0, w)],
                sis[p],
            ).start()

    def loadwait(p, w):
        pltpu.make_async_copy(
            wt.at[pl.ds(0, DIM), pl.ds(0, w)], ins[p].at[:, pl.ds(0, w)], sis[p]
        ).wait()

    for j in range(DIM):
        dv = (j // L) * L + ((lanes + j) % L)
        dtab[pl.ds(j * L, L)] = dv.astype(jnp.int32)

    def transpose(p, n):
        @pl.loop(0, n // L)
        def _(rq):
            rows = rq * L + lanes
            for j0 in range(0, DIM, 4):
                dvs = [dtab[pl.ds((j0 + i) * L, L)] for i in range(4)]
                vecs = [plsc.load_gather(ins[p], [dvs[i], rows]) for i in range(4)]
                for i in range(4):
                    plsc.store_scatter(ots[p], [rows, dvs[i]], vecs[i])

    def store(g, p, n):
        pltpu.make_async_copy(
            ots[p].at[pl.ds(0, n), :], tblp.at[pl.ds(g * GCOLS, n), :], sos[p]
        ).start()

    def storewait(p, n):
        pltpu.make_async_copy(
            ots[p].at[pl.ds(0, n), :], tblp.at[pl.ds(0, n), :], sos[p]
        ).wait()

    def body(g, p):
        @pl.when(g + 1 < g_hi)
        def _():
            load(g + 1, 1 - p, GCOLS)

        loadwait(p, GCOLS)

        @pl.when(g - g_lo >= 2)
        def _():
            storewait(p, GCOLS)

        transpose(p, GCOLS)
        store(g, p, GCOLS)

    load(g_lo, 0, GCOLS)

    @pl.loop(0, cnt // 2)
    def _(jp):
        g0 = g_lo + 2 * jp
        body(g0, 0)
        body(g0 + 1, 1)

    @pl.when(cnt % 2 == 1)
    def _():
        body(g_hi - 1, 0)

    storewait(0, GCOLS)
    storewait(1, GCOLS)

    @pl.when(wid == NW - 1)
    def _():
        gt = jnp.int32(NGRP)
        load(gt, 0, PDIM)
        loadwait(0, PDIM)
        transpose(0, TAIL)
        store(gt, 0, TAIL)
        storewait(0, TAIL)


@functools.partial(
    pl.kernel,
    out_type=jax.ShapeDtypeStruct((BATCH * FPAD, PDIM), jnp.float32),
    mesh=_mesh,
    scratch_types=[
        pltpu.VMEM((LPW,), jnp.int32),
        pltpu.VMEM((CHUNK, PDIM), jnp.float32),
        pltpu.VMEM((CHUNK, PDIM), jnp.float32),
        pltpu.SemaphoreType.DMA,
        pltpu.SemaphoreType.DMA,
    ],
    compiler_params=pltpu.CompilerParams(use_tc_tiling_on_sc=False),
)
def _sc_gather(tbl, idx, out, idx_v, rows0, rows1, sem0, sem1):
    wid = lax.axis_index("s") * NC + lax.axis_index("c")
    base = wid * LPW
    pltpu.sync_copy(idx.at[pl.ds(base, LPW)], idx_v)

    bufs = (rows0, rows1)
    sems = (sem0, sem1)

    def start(c, b):
        pltpu.make_async_copy(
            tbl.at[idx_v.at[pl.ds(c * CHUNK, CHUNK)]], bufs[b], sems[b]
        ).start()

    def finish(c, b):
        pltpu.make_async_copy(
            tbl.at[idx_v.at[pl.ds(c * CHUNK, CHUNK)]], bufs[b], sems[b]
        ).wait()
        b0 = wid * BPW + c * SUB
        for j in range(SUB):
            pltpu.sync_copy(
                bufs[b].at[pl.ds(j * FIELDS, FIELDS), :],
                out.at[pl.ds((b0 + j) * FPAD, FIELDS), :],
            )

    start(0, 0)
    start(1, 1)

    @pl.loop(0, NCHUNK // 2 - 1)
    def _(g):
        for b in range(2):
            c = 2 * g + b
            finish(c, b)
            start(c + 2, b)

    for b in range(2):
        finish(NCHUNK - 2 + b, b)


def kernel(input, weight):
    idx = input.reshape(-1).astype(jnp.int32)
    tblp = _sc_repack(weight.T)
    out = _sc_gather(tblp, idx)
    return out.reshape(BATCH, FPAD, PDIM)[:, :FIELDS, :DIM]


# speedup vs baseline: 2.1237x; 1.0619x over previous
import jax
import jax.numpy as jnp
from jax import lax
from jax.experimental import pallas as pl
from jax.experimental.pallas import tpu as pltpu
from jax.experimental.pallas import tpu_sc as plsc
import functools

NUM_EMB = 1_000_000
DIM = 64
PDIM = 128
BATCH = 16384
FIELDS = 26
FPAD = 32
TOT = BATCH * FIELDS

NC = 2
NS = 16
NW = NC * NS
L = 16

GCOLS = 256
NGRP = NUM_EMB // GCOLS
TAIL = NUM_EMB - NGRP * GCOLS
assert TAIL == 64

BPW = BATCH // NW
LPW = BPW * FIELDS
SUB = 16
CHUNK = SUB * FIELDS
NCHUNK = LPW // CHUNK
assert NCHUNK % 2 == 0

_mesh = plsc.VectorSubcoreMesh(
    core_axis_name="c", subcore_axis_name="s", num_cores=NC, num_subcores=NS
)


@functools.partial(
    pl.kernel,
    out_type=jax.ShapeDtypeStruct((NUM_EMB // 2, PDIM), jnp.float32),
    mesh=_mesh,
    scratch_types=[
        pltpu.VMEM((DIM, GCOLS), jnp.float32),
        pltpu.VMEM((DIM, GCOLS), jnp.float32),
        pltpu.VMEM((GCOLS // 2, PDIM), jnp.float32),
        pltpu.VMEM((GCOLS // 2, PDIM), jnp.float32),
        pltpu.VMEM((DIM * L,), jnp.int32),
        pltpu.SemaphoreType.DMA,
        pltpu.SemaphoreType.DMA,
        pltpu.SemaphoreType.DMA,
        pltpu.SemaphoreType.DMA,
    ],
    compiler_params=pltpu.CompilerParams(needs_layout_passes=False),
)
def _sc_repack(wt, tblp, in0, in1, ot0, ot1, dtab, si0, si1, so0, so1):
    wid = lax.axis_index("s") * NC + lax.axis_index("c")
    ins = (in0, in1)
    ots = (ot0, ot1)
    sis = (si0, si1)
    sos = (so0, so1)
    lanes = lax.iota(jnp.int32, L)

    g_lo = wid * NGRP // NW
    g_hi = (wid + 1) * NGRP // NW
    cnt = g_hi - g_lo

    def load(g, p, w):
        @pl.loop(0, DIM // 8)
        def _(t):
            pltpu.make_async_copy(
                wt.at[pl.ds(t * 8, 8), pl.ds(g * GCOLS, w)],
                ins[p].at[pl.ds(t * 8, 8), pl.ds(0, w)],
                sis[p],
            ).start()

    def loadwait(p, w):
        pltpu.make_async_copy(
            wt.at[pl.ds(0, DIM), pl.ds(0, w)], ins[p].at[:, pl.ds(0, w)], sis[p]
        ).wait()

    for j in range(DIM):
        dv = (j // L) * L + ((lanes + j) % L)
        dtab[pl.ds(j * L, L)] = dv.astype(jnp.int32)

    def transpose(p, n):
        @pl.loop(0, n // L)
        def _(rq):
            rows = rq * L + lanes
            qrow = lax.shift_right_logical(rows, 1)
            hoff = (rows & 1) * DIM
            for j0 in range(0, DIM, 4):
                dvs = [dtab[pl.ds((j0 + i) * L, L)] for i in range(4)]
                vecs = [plsc.load_gather(ins[p], [dvs[i], rows]) for i in range(4)]
                for i in range(4):
                    plsc.store_scatter(ots[p], [qrow, hoff + dvs[i]], vecs[i])

    def store(g, p, n):
        pltpu.make_async_copy(
            ots[p].at[pl.ds(0, n // 2), :],
            tblp.at[pl.ds(g * (GCOLS // 2), n // 2), :],
            sos[p],
        ).start()

    def storewait(p, n):
        pltpu.make_async_copy(
            ots[p].at[pl.ds(0, n // 2), :], tblp.at[pl.ds(0, n // 2), :], sos[p]
        ).wait()

    def body(g, p):
        @pl.when(g + 1 < g_hi)
        def _():
            load(g + 1, 1 - p, GCOLS)

        loadwait(p, GCOLS)

        @pl.when(g - g_lo >= 2)
        def _():
            storewait(p, GCOLS)

        transpose(p, GCOLS)
        store(g, p, GCOLS)

    load(g_lo, 0, GCOLS)

    @pl.loop(0, cnt // 2)
    def _(jp):
        g0 = g_lo + 2 * jp
        body(g0, 0)
        body(g0 + 1, 1)

    @pl.when(cnt % 2 == 1)
    def _():
        body(g_hi - 1, 0)

    storewait(0, GCOLS)
    storewait(1, GCOLS)

    @pl.when(wid == NW - 1)
    def _():
        gt = jnp.int32(NGRP)
        load(gt, 0, PDIM)
        loadwait(0, PDIM)
        transpose(0, TAIL)
        store(gt, 0, TAIL)
        storewait(0, TAIL)


@functools.partial(
    pl.kernel,
    out_type=jax.ShapeDtypeStruct((BATCH * FPAD, PDIM), jnp.float32),
    mesh=_mesh,
    scratch_types=[
        pltpu.VMEM((LPW,), jnp.int32),
        pltpu.VMEM((CHUNK, DIM), jnp.float32),
        pltpu.VMEM((CHUNK, DIM), jnp.float32),
        pltpu.SemaphoreType.DMA,
        pltpu.SemaphoreType.DMA,
    ],
    compiler_params=pltpu.CompilerParams(use_tc_tiling_on_sc=False),
)
def _sc_gather(tbl, idx, out, idx_v, rows0, rows1, sem0, sem1):
    wid = lax.axis_index("s") * NC + lax.axis_index("c")
    base = wid * LPW
    pltpu.sync_copy(idx.at[pl.ds(base, LPW)], idx_v)

    bufs = (rows0, rows1)
    sems = (sem0, sem1)

    def start(c, b):
        pltpu.make_async_copy(
            tbl.at[idx_v.at[pl.ds(c * CHUNK, CHUNK)]], bufs[b], sems[b]
        ).start()

    def finish(c, b):
        pltpu.make_async_copy(
            tbl.at[idx_v.at[pl.ds(c * CHUNK, CHUNK)]], bufs[b], sems[b]
        ).wait()
        b0 = wid * BPW + c * SUB
        for j in range(SUB):
            pltpu.sync_copy(
                bufs[b].at[pl.ds(j * FIELDS, FIELDS), :],
                out.at[pl.ds((b0 + j) * FPAD, FIELDS), pl.ds(0, DIM)],
            )

    start(0, 0)
    start(1, 1)

    @pl.loop(0, NCHUNK // 2 - 1)
    def _(g):
        for b in range(2):
            c = 2 * g + b
            finish(c, b)
            start(c + 2, b)

    for b in range(2):
        finish(NCHUNK - 2 + b, b)


def kernel(input, weight):
    idx = input.reshape(-1).astype(jnp.int32)
    tblp = _sc_repack(weight.T).reshape(NUM_EMB, DIM)
    out = _sc_gather(tblp, idx)
    return out.reshape(BATCH, FPAD, PDIM)[:, :FIELDS, :DIM]


# speedup vs baseline: 2.5061x; 1.1801x over previous
import jax
import jax.numpy as jnp
from jax import lax
from jax.experimental import pallas as pl
from jax.experimental.pallas import tpu as pltpu
from jax.experimental.pallas import tpu_sc as plsc
import functools

NUM_EMB = 1_000_000
DIM = 64
PDIM = 128
BATCH = 16384
FIELDS = 26
FPAD = 32
TOT = BATCH * FIELDS

NC = 2
NS = 16
NW = NC * NS
L = 16

GCOLS = 256
NGRP = NUM_EMB // GCOLS
TAIL = NUM_EMB - NGRP * GCOLS
assert TAIL == 64

BPW = BATCH // NW
LPW = BPW * FIELDS
SUB = 16
CHUNK = SUB * FIELDS
NCHUNK = LPW // CHUNK
assert NCHUNK % 2 == 0

_mesh = plsc.VectorSubcoreMesh(
    core_axis_name="c", subcore_axis_name="s", num_cores=NC, num_subcores=NS
)


@functools.partial(
    pl.kernel,
    out_type=jax.ShapeDtypeStruct((NUM_EMB // 2, PDIM), jnp.float32),
    mesh=_mesh,
    scratch_types=[
        pltpu.VMEM((DIM, GCOLS), jnp.float32),
        pltpu.VMEM((DIM, GCOLS), jnp.float32),
        pltpu.VMEM((GCOLS // 2, PDIM), jnp.float32),
        pltpu.VMEM((GCOLS // 2, PDIM), jnp.float32),
        pltpu.VMEM((2 * DIM * L,), jnp.int32),
        pltpu.SemaphoreType.DMA,
        pltpu.SemaphoreType.DMA,
        pltpu.SemaphoreType.DMA,
        pltpu.SemaphoreType.DMA,
    ],
    compiler_params=pltpu.CompilerParams(needs_layout_passes=False),
)
def _sc_repack(wt, tblp, in0, in1, ot0, ot1, dtab, si0, si1, so0, so1):
    wid = lax.axis_index("s") * NC + lax.axis_index("c")
    ins = (in0, in1)
    ots = (ot0, ot1)
    sis = (si0, si1)
    sos = (so0, so1)
    lanes = lax.iota(jnp.int32, L)

    g_lo = wid * NGRP // NW
    g_hi = (wid + 1) * NGRP // NW
    cnt = g_hi - g_lo

    def load(g, p, w):
        @pl.loop(0, DIM // 8)
        def _(t):
            pltpu.make_async_copy(
                wt.at[pl.ds(t * 8, 8), pl.ds(g * GCOLS, w)],
                ins[p].at[pl.ds(t * 8, 8), pl.ds(0, w)],
                sis[p],
            ).start()

    def loadwait(p, w):
        pltpu.make_async_copy(
            wt.at[pl.ds(0, DIM), pl.ds(0, w)], ins[p].at[:, pl.ds(0, w)], sis[p]
        ).wait()

    for j in range(DIM):
        dv = (j // L) * L + ((lanes + j) % L)
        dtab[pl.ds(j * L, L)] = dv.astype(jnp.int32)
        dtab[pl.ds((DIM + j) * L, L)] = (dv + (lanes % 2) * DIM).astype(jnp.int32)

    def transpose(p, n):
        @pl.loop(0, n // L)
        def _(rq):
            rows = rq * L + lanes
            qrow = rq * (L // 2) + lax.shift_right_logical(lanes, 1)
            for j0 in range(0, DIM, 8):
                dvs = [dtab[pl.ds((j0 + i) * L, L)] for i in range(8)]
                cvs = [dtab[pl.ds((DIM + j0 + i) * L, L)] for i in range(8)]
                vecs = [plsc.load_gather(ins[p], [dvs[i], rows]) for i in range(8)]
                for i in range(8):
                    plsc.store_scatter(ots[p], [qrow, cvs[i]], vecs[i])

    def store(g, p, n):
        pltpu.make_async_copy(
            ots[p].at[pl.ds(0, n // 2), :],
            tblp.at[pl.ds(g * (GCOLS // 2), n // 2), :],
            sos[p],
        ).start()

    def storewait(p, n):
        pltpu.make_async_copy(
            ots[p].at[pl.ds(0, n // 2), :], tblp.at[pl.ds(0, n // 2), :], sos[p]
        ).wait()

    def body(g, p):
        @pl.when(g + 1 < g_hi)
        def _():
            load(g + 1, 1 - p, GCOLS)

        loadwait(p, GCOLS)

        @pl.when(g - g_lo >= 2)
        def _():
            storewait(p, GCOLS)

        transpose(p, GCOLS)
        store(g, p, GCOLS)

    load(g_lo, 0, GCOLS)

    @pl.loop(0, cnt // 2)
    def _(jp):
        g0 = g_lo + 2 * jp
        body(g0, 0)
        body(g0 + 1, 1)

    @pl.when(cnt % 2 == 1)
    def _():
        body(g_hi - 1, 0)

    storewait(0, GCOLS)
    storewait(1, GCOLS)

    @pl.when(wid == NW - 1)
    def _():
        gt = jnp.int32(NGRP)
        load(gt, 0, PDIM)
        loadwait(0, PDIM)
        transpose(0, TAIL)
        store(gt, 0, TAIL)
        storewait(0, TAIL)


@functools.partial(
    pl.kernel,
    out_type=jax.ShapeDtypeStruct((BATCH * FPAD, PDIM), jnp.float32),
    mesh=_mesh,
    scratch_types=[
        pltpu.VMEM((LPW,), jnp.int32),
        pltpu.VMEM((CHUNK, DIM), jnp.float32),
        pltpu.VMEM((CHUNK, DIM), jnp.float32),
        pltpu.SemaphoreType.DMA,
        pltpu.SemaphoreType.DMA,
    ],
    compiler_params=pltpu.CompilerParams(use_tc_tiling_on_sc=False),
)
def _sc_gather(tbl, idx, out, idx_v, rows0, rows1, sem0, sem1):
    wid = lax.axis_index("s") * NC + lax.axis_index("c")
    base = wid * LPW
    pltpu.sync_copy(idx.at[pl.ds(base, LPW)], idx_v)

    bufs = (rows0, rows1)
    sems = (sem0, sem1)

    def start(c, b):
        pltpu.make_async_copy(
            tbl.at[idx_v.at[pl.ds(c * CHUNK, CHUNK)]], bufs[b], sems[b]
        ).start()

    def finish(c, b):
        pltpu.make_async_copy(
            tbl.at[idx_v.at[pl.ds(c * CHUNK, CHUNK)]], bufs[b], sems[b]
        ).wait()
        b0 = wid * BPW + c * SUB
        for j in range(SUB):
            pltpu.sync_copy(
                bufs[b].at[pl.ds(j * FIELDS, FIELDS), :],
                out.at[pl.ds((b0 + j) * FPAD, FIELDS), pl.ds(0, DIM)],
            )

    start(0, 0)
    start(1, 1)

    @pl.loop(0, NCHUNK // 2 - 1)
    def _(g):
        for b in range(2):
            c = 2 * g + b
            finish(c, b)
            start(c + 2, b)

    for b in range(2):
        finish(NCHUNK - 2 + b, b)


def kernel(input, weight):
    idx = input.reshape(-1).astype(jnp.int32)
    tblp = _sc_repack(weight.T).reshape(NUM_EMB, DIM)
    out = _sc_gather(tblp, idx)
    return out.reshape(BATCH, FPAD, PDIM)[:, :FIELDS, :DIM]


# speedup vs baseline: 3.7403x; 1.4925x over previous
import jax
import jax.numpy as jnp
from jax import lax
from jax.experimental import pallas as pl
from jax.experimental.pallas import tpu as pltpu
from jax.experimental.pallas import tpu_sc as plsc
import functools

NUM_EMB = 1_000_000
DIM = 64
PDIM = 128
BATCH = 16384
FIELDS = 26
FPAD = 32
TOT = BATCH * FIELDS

NC = 2
NS = 16
NW = NC * NS
L = 16

GCOLS = 256
NGRP = NUM_EMB // GCOLS
TAIL = NUM_EMB - NGRP * GCOLS
assert TAIL == 64

BPW = BATCH // NW
LPW = BPW * FIELDS
SUB = 16
CHUNK = SUB * FIELDS
NCHUNK = LPW // CHUNK
assert NCHUNK % 2 == 0

_mesh = plsc.VectorSubcoreMesh(
    core_axis_name="c", subcore_axis_name="s", num_cores=NC, num_subcores=NS
)


@functools.partial(
    pl.kernel,
    out_type=jax.ShapeDtypeStruct((NUM_EMB // 2, PDIM), jnp.float32),
    mesh=_mesh,
    scratch_types=[
        pltpu.VMEM((DIM, GCOLS), jnp.float32),
        pltpu.VMEM((DIM, GCOLS), jnp.float32),
        pltpu.VMEM((GCOLS // 2, PDIM), jnp.float32),
        pltpu.VMEM((GCOLS // 2, PDIM), jnp.float32),
        pltpu.VMEM((2 * DIM * L,), jnp.int32),
        pltpu.SemaphoreType.DMA,
        pltpu.SemaphoreType.DMA,
        pltpu.SemaphoreType.DMA,
        pltpu.SemaphoreType.DMA,
    ],
    compiler_params=pltpu.CompilerParams(needs_layout_passes=False),
)
def _sc_repack(wt, tblp, in0, in1, ot0, ot1, dtab, si0, si1, so0, so1):
    wid = lax.axis_index("s") * NC + lax.axis_index("c")
    ins = (in0, in1)
    ots = (ot0, ot1)
    sis = (si0, si1)
    sos = (so0, so1)
    lanes = lax.iota(jnp.int32, L)

    g_lo = wid * NGRP // NW
    g_hi = (wid + 1) * NGRP // NW
    cnt = g_hi - g_lo

    def load(g, p, w):
        @pl.loop(0, DIM // 8)
        def _(t):
            pltpu.make_async_copy(
                wt.at[pl.ds(t * 8, 8), pl.ds(g * GCOLS, w)],
                ins[p].at[pl.ds(t * 8, 8), pl.ds(0, w)],
                sis[p],
            ).start()

    def loadwait(p, w):
        pltpu.make_async_copy(
            wt.at[pl.ds(0, DIM), pl.ds(0, w)], ins[p].at[:, pl.ds(0, w)], sis[p]
        ).wait()

    for j in range(DIM):
        dv = (j // L) * L + ((lanes + j) % L)
        dtab[pl.ds(j * L, L)] = dv.astype(jnp.int32)
        dtab[pl.ds((DIM + j) * L, L)] = (dv + (lanes % 2) * DIM).astype(jnp.int32)

    def transpose(p, n):
        hlane = lax.shift_right_logical(lanes, 1)
        for j0 in range(0, DIM, 8):
            dvs = [dtab[pl.ds((j0 + i) * L, L)] for i in range(8)]
            cvs = [dtab[pl.ds((DIM + j0 + i) * L, L)] for i in range(8)]

            @pl.loop(0, n // L)
            def _(rq):
                rows = rq * L + lanes
                qrow = rq * (L // 2) + hlane
                vecs = [plsc.load_gather(ins[p], [dvs[i], rows]) for i in range(8)]
                for i in range(8):
                    plsc.store_scatter(ots[p], [qrow, cvs[i]], vecs[i])

    def store(g, p, n):
        pltpu.make_async_copy(
            ots[p].at[pl.ds(0, n // 2), :],
            tblp.at[pl.ds(g * (GCOLS // 2), n // 2), :],
            sos[p],
        ).start()

    def storewait(p, n):
        pltpu.make_async_copy(
            ots[p].at[pl.ds(0, n // 2), :], tblp.at[pl.ds(0, n // 2), :], sos[p]
        ).wait()

    def body(g, p):
        @pl.when(g + 1 < g_hi)
        def _():
            load(g + 1, 1 - p, GCOLS)

        loadwait(p, GCOLS)

        @pl.when(g - g_lo >= 2)
        def _():
            storewait(p, GCOLS)

        transpose(p, GCOLS)
        store(g, p, GCOLS)

    load(g_lo, 0, GCOLS)

    @pl.loop(0, cnt // 2)
    def _(jp):
        g0 = g_lo + 2 * jp
        body(g0, 0)
        body(g0 + 1, 1)

    @pl.when(cnt % 2 == 1)
    def _():
        body(g_hi - 1, 0)

    storewait(0, GCOLS)
    storewait(1, GCOLS)

    @pl.when(wid == NW - 1)
    def _():
        gt = jnp.int32(NGRP)
        load(gt, 0, PDIM)
        loadwait(0, PDIM)
        transpose(0, TAIL)
        store(gt, 0, TAIL)
        storewait(0, TAIL)


@functools.partial(
    pl.kernel,
    out_type=jax.ShapeDtypeStruct((BATCH * FPAD, PDIM), jnp.float32),
    mesh=_mesh,
    scratch_types=[
        pltpu.VMEM((LPW,), jnp.int32),
        pltpu.VMEM((CHUNK, DIM), jnp.float32),
        pltpu.VMEM((CHUNK, DIM), jnp.float32),
        pltpu.SemaphoreType.DMA,
        pltpu.SemaphoreType.DMA,
    ],
    compiler_params=pltpu.CompilerParams(use_tc_tiling_on_sc=False),
)
def _sc_gather(tbl, idx, out, idx_v, rows0, rows1, sem0, sem1):
    wid = lax.axis_index("s") * NC + lax.axis_index("c")
    base = wid * LPW
    pltpu.sync_copy(idx.at[pl.ds(base, LPW)], idx_v)

    bufs = (rows0, rows1)
    sems = (sem0, sem1)

    def start(c, b):
        pltpu.make_async_copy(
            tbl.at[idx_v.at[pl.ds(c * CHUNK, CHUNK)]], bufs[b], sems[b]
        ).start()

    def finish(c, b):
        pltpu.make_async_copy(
            tbl.at[idx_v.at[pl.ds(c * CHUNK, CHUNK)]], bufs[b], sems[b]
        ).wait()
        b0 = wid * BPW + c * SUB
        for j in range(SUB):
            pltpu.sync_copy(
                bufs[b].at[pl.ds(j * FIELDS, FIELDS), :],
                out.at[pl.ds((b0 + j) * FPAD, FIELDS), pl.ds(0, DIM)],
            )

    start(0, 0)
    start(1, 1)

    @pl.loop(0, NCHUNK // 2 - 1)
    def _(g):
        for b in range(2):
            c = 2 * g + b
            finish(c, b)
            start(c + 2, b)

    for b in range(2):
        finish(NCHUNK - 2 + b, b)


def kernel(input, weight):
    idx = input.reshape(-1).astype(jnp.int32)
    tblp = _sc_repack(weight.T).reshape(NUM_EMB, DIM)
    out = _sc_gather(tblp, idx)
    return out.reshape(BATCH, FPAD, PDIM)[:, :FIELDS, :DIM]


# speedup vs baseline: 3.8949x; 1.0413x over previous
import jax
import jax.numpy as jnp
from jax import lax
from jax.experimental import pallas as pl
from jax.experimental.pallas import tpu as pltpu
from jax.experimental.pallas import tpu_sc as plsc
import functools

NUM_EMB = 1_000_000
DIM = 64
PDIM = 128
BATCH = 16384
FIELDS = 26
FPAD = 32
TOT = BATCH * FIELDS

NC = 2
NS = 16
NW = NC * NS
L = 16

GCOLS = 384
NGRP = NUM_EMB // GCOLS
TAIL = NUM_EMB - NGRP * GCOLS
assert TAIL == 64

BPW = BATCH // NW
LPW = BPW * FIELDS
SUB = 16
CHUNK = SUB * FIELDS
NCHUNK = LPW // CHUNK
assert NCHUNK % 2 == 0

_mesh = plsc.VectorSubcoreMesh(
    core_axis_name="c", subcore_axis_name="s", num_cores=NC, num_subcores=NS
)


@functools.partial(
    pl.kernel,
    out_type=jax.ShapeDtypeStruct((NUM_EMB // 2, PDIM), jnp.float32),
    mesh=_mesh,
    scratch_types=[
        pltpu.VMEM((DIM, GCOLS), jnp.float32),
        pltpu.VMEM((DIM, GCOLS), jnp.float32),
        pltpu.VMEM((GCOLS // 2, PDIM), jnp.float32),
        pltpu.VMEM((GCOLS // 2, PDIM), jnp.float32),
        pltpu.VMEM((2 * DIM * L,), jnp.int32),
        pltpu.SemaphoreType.DMA,
        pltpu.SemaphoreType.DMA,
        pltpu.SemaphoreType.DMA,
        pltpu.SemaphoreType.DMA,
    ],
    compiler_params=pltpu.CompilerParams(needs_layout_passes=False),
)
def _sc_repack(wt, tblp, in0, in1, ot0, ot1, dtab, si0, si1, so0, so1):
    wid = lax.axis_index("s") * NC + lax.axis_index("c")
    ins = (in0, in1)
    ots = (ot0, ot1)
    sis = (si0, si1)
    sos = (so0, so1)
    lanes = lax.iota(jnp.int32, L)

    g_lo = wid * NGRP // NW
    g_hi = (wid + 1) * NGRP // NW
    cnt = g_hi - g_lo

    def load(g, p, w):
        @pl.loop(0, DIM // 8)
        def _(t):
            pltpu.make_async_copy(
                wt.at[pl.ds(t * 8, 8), pl.ds(g * GCOLS, w)],
                ins[p].at[pl.ds(t * 8, 8), pl.ds(0, w)],
                sis[p],
            ).start()

    def loadwait(p, w):
        pltpu.make_async_copy(
            wt.at[pl.ds(0, DIM), pl.ds(0, w)], ins[p].at[:, pl.ds(0, w)], sis[p]
        ).wait()

    for j in range(DIM):
        dv = (j // L) * L + ((lanes + j) % L)
        dtab[pl.ds(j * L, L)] = dv.astype(jnp.int32)
        dtab[pl.ds((DIM + j) * L, L)] = (dv + (lanes % 2) * DIM).astype(jnp.int32)

    def transpose(p, n):
        hlane = lax.shift_right_logical(lanes, 1)
        for j0 in range(0, DIM, 8):
            dvs = [dtab[pl.ds((j0 + i) * L, L)] for i in range(8)]
            cvs = [dtab[pl.ds((DIM + j0 + i) * L, L)] for i in range(8)]

            @pl.loop(0, n // L)
            def _(rq):
                rows = rq * L + lanes
                qrow = rq * (L // 2) + hlane
                vecs = [plsc.load_gather(ins[p], [dvs[i], rows]) for i in range(8)]
                for i in range(8):
                    plsc.store_scatter(ots[p], [qrow, cvs[i]], vecs[i])

    def store(g, p, n):
        pltpu.make_async_copy(
            ots[p].at[pl.ds(0, n // 2), :],
            tblp.at[pl.ds(g * (GCOLS // 2), n // 2), :],
            sos[p],
        ).start()

    def storewait(p, n):
        pltpu.make_async_copy(
            ots[p].at[pl.ds(0, n // 2), :], tblp.at[pl.ds(0, n // 2), :], sos[p]
        ).wait()

    def body(g, p):
        @pl.when(g + 1 < g_hi)
        def _():
            load(g + 1, 1 - p, GCOLS)

        loadwait(p, GCOLS)

        @pl.when(g - g_lo >= 2)
        def _():
            storewait(p, GCOLS)

        transpose(p, GCOLS)
        store(g, p, GCOLS)

    load(g_lo, 0, GCOLS)

    @pl.loop(0, cnt // 2)
    def _(jp):
        g0 = g_lo + 2 * jp
        body(g0, 0)
        body(g0 + 1, 1)

    @pl.when(cnt % 2 == 1)
    def _():
        body(g_hi - 1, 0)

    storewait(0, GCOLS)
    storewait(1, GCOLS)

    @pl.when(wid == NW - 1)
    def _():
        gt = jnp.int32(NGRP)
        load(gt, 0, PDIM)
        loadwait(0, PDIM)
        transpose(0, TAIL)
        store(gt, 0, TAIL)
        storewait(0, TAIL)


@functools.partial(
    pl.kernel,
    out_type=jax.ShapeDtypeStruct((BATCH * FPAD, PDIM), jnp.float32),
    mesh=_mesh,
    scratch_types=[
        pltpu.VMEM((LPW,), jnp.int32),
        pltpu.VMEM((CHUNK, DIM), jnp.float32),
        pltpu.VMEM((CHUNK, DIM), jnp.float32),
        pltpu.SemaphoreType.DMA,
        pltpu.SemaphoreType.DMA,
    ],
    compiler_params=pltpu.CompilerParams(use_tc_tiling_on_sc=False),
)
def _sc_gather(tbl, idx, out, idx_v, rows0, rows1, sem0, sem1):
    wid = lax.axis_index("s") * NC + lax.axis_index("c")
    base = wid * LPW
    pltpu.sync_copy(idx.at[pl.ds(base, LPW)], idx_v)

    bufs = (rows0, rows1)
    sems = (sem0, sem1)

    def start(c, b):
        pltpu.make_async_copy(
            tbl.at[idx_v.at[pl.ds(c * CHUNK, CHUNK)]], bufs[b], sems[b]
        ).start()

    def finish(c, b):
        pltpu.make_async_copy(
            tbl.at[idx_v.at[pl.ds(c * CHUNK, CHUNK)]], bufs[b], sems[b]
        ).wait()
        b0 = wid * BPW + c * SUB
        for j in range(SUB):
            pltpu.sync_copy(
                bufs[b].at[pl.ds(j * FIELDS, FIELDS), :],
                out.at[pl.ds((b0 + j) * FPAD, FIELDS), pl.ds(0, DIM)],
            )

    start(0, 0)
    start(1, 1)

    @pl.loop(0, NCHUNK // 2 - 1)
    def _(g):
        for b in range(2):
            c = 2 * g + b
            finish(c, b)
            start(c + 2, b)

    for b in range(2):
        finish(NCHUNK - 2 + b, b)


def kernel(input, weight):
    idx = input.reshape(-1).astype(jnp.int32)
    tblp = _sc_repack(weight.T).reshape(NUM_EMB, DIM)
    out = _sc_gather(tblp, idx)
    return out.reshape(BATCH, FPAD, PDIM)[:, :FIELDS, :DIM]
